# Initial kernel scaffold; baseline (speedup 1.0000x reference)
import jax
import jax.numpy as jnp
from jax import lax
from jax.experimental import pallas as pl
from jax.experimental.pallas import tpu as pltpu
from jax.experimental.pallas import tpu_sc as plsc

N = 10000
D = 256
H = 128
NPAD = 10240
E2 = 170000
EPAD = 170240
ET = EPAD // 16
NBATCH = ET // 16
ROWS_PT = NPAD // 16
DBLK = 64
DEN_ROWS = NPAD // H
NB = 512
f32 = jnp.float32


def _mm1_body(x_ref, w_ref, a2_ref, h2_ref, sd_ref):
    c = pl.program_id(1)
    h = jnp.dot(x_ref[...], w_ref[...], preferred_element_type=f32)
    h2_ref[0] = h
    part = jnp.dot(h, a2_ref[...], preferred_element_type=f32)

    @pl.when(c == 0)
    def _():
        sd_ref[...] = part

    @pl.when(c == 1)
    def _():
        sd_ref[...] = sd_ref[...] + part


def _layer1_matmul(x_p, W, A2p):
    return pl.pallas_call(
        _mm1_body,
        grid=(NPAD // NB, 2),
        in_specs=[
            pl.BlockSpec((NB, D), lambda i, c: (i, 0)),
            pl.BlockSpec((D, H), lambda i, c: (0, c)),
            pl.BlockSpec((H, H), lambda i, c: (c, 0)),
        ],
        out_specs=[
            pl.BlockSpec((1, NB, H), lambda i, c: (c, i, 0)),
            pl.BlockSpec((NB, H), lambda i, c: (i, 0)),
        ],
        out_shape=[
            jax.ShapeDtypeStruct((2, NPAD, H), f32),
            jax.ShapeDtypeStruct((NPAD, H), f32),
        ],
    )(x_p, W, A2p)


def _mm23_body(z_ref, b_ref, w_ref, a2_ref, h2_ref, sd_ref):
    c = pl.program_id(1)
    act0 = jnp.maximum(z_ref[0] + b_ref[0], 0.0)
    act1 = jnp.maximum(z_ref[1] + b_ref[1], 0.0)
    h = (jnp.dot(act0, w_ref[0], preferred_element_type=f32)
         + jnp.dot(act1, w_ref[1], preferred_element_type=f32))
    h2_ref[0] = h
    part = jnp.dot(h, a2_ref[...], preferred_element_type=f32)

    @pl.when(c == 0)
    def _():
        sd_ref[...] = part

    @pl.when(c == 1)
    def _():
        sd_ref[...] = sd_ref[...] + part


def _layer23_matmul(z, b2d, W3d, A2p):
    return pl.pallas_call(
        _mm23_body,
        grid=(NPAD // NB, 2),
        in_specs=[
            pl.BlockSpec((2, NB, H), lambda i, c: (0, i, 0)),
            pl.BlockSpec((2, H), lambda i, c: (0, 0)),
            pl.BlockSpec((2, H, H), lambda i, c: (0, 0, c)),
            pl.BlockSpec((H, H), lambda i, c: (c, 0)),
        ],
        out_specs=[
            pl.BlockSpec((1, NB, H), lambda i, c: (c, i, 0)),
            pl.BlockSpec((NB, H), lambda i, c: (i, 0)),
        ],
        out_shape=[
            jax.ShapeDtypeStruct((2, NPAD, H), f32),
            jax.ShapeDtypeStruct((NPAD, H), f32),
        ],
    )(z, b2d, W3d, A2p)


def _lsm_body(z_ref, b_ref, o_ref):
    u0 = z_ref[0] + b_ref[0]
    u1 = z_ref[1] + b_ref[1]
    u = jnp.concatenate([u0, u1], axis=1)
    m = jnp.max(u, axis=1, keepdims=True)
    lse = jnp.log(jnp.sum(jnp.exp(u - m), axis=1, keepdims=True))
    o_ref[...] = u - m - lse


def _log_softmax(z, b2d):
    return pl.pallas_call(
        _lsm_body,
        grid=(NPAD // NB,),
        in_specs=[
            pl.BlockSpec((2, NB, H), lambda i: (0, i, 0)),
            pl.BlockSpec((2, H), lambda i: (0, 0)),
        ],
        out_specs=pl.BlockSpec((NB, D), lambda i: (i, 0)),
        out_shape=jax.ShapeDtypeStruct((NPAD, D), f32),
    )(z, b2d)


def _gat_sc_body(h2f, s_hbm, d_hbm, src_hbm, dst_hbm,
                 z_hbm,
                 s_v, d_v, den_v, src_v, dst_v, rows_v, wbuf_v, idx80_v,
                 dch_v, zbuf_v, y_sh, den_sh, sem_g, sem_s):
    cid = lax.axis_index("c")
    wid = lax.axis_index("s")
    base_e = wid * ET
    row0 = wid * ROWS_PT
    cN = cid * NPAD

    pltpu.sync_copy(s_hbm, s_v)
    pltpu.sync_copy(d_hbm, d_v)
    pltpu.sync_copy(src_hbm.at[pl.ds(base_e, ET)], src_v)
    pltpu.sync_copy(dst_hbm.at[pl.ds(base_e, ET)], dst_v)

    zeros16 = jnp.zeros((16,), f32)

    def _zden(i, carry):
        den_v[lax.div(i, 8), pl.ds(lax.rem(i, 8) * 16, 16)] = zeros16
        return carry
    lax.fori_loop(0, DEN_ROWS * 8, _zden, 0)

    def _zbuf(i, carry):
        zbuf_v[lax.div(i, 8), pl.ds(lax.rem(i, 8) * 16, 16)] = zeros16
        return carry
    lax.fori_loop(0, DBLK * 8, _zbuf, 0)

    def _iot(i, carry):
        idx80_v[pl.ds(i * 16, 16)] = lax.iota(jnp.int32, 16) + i * 16
        return carry
    lax.fori_loop(0, DEN_ROWS // 16, _iot, 0)

    def _zy(kblk, carry):
        pltpu.sync_copy(zbuf_v, y_sh.at[pl.ds(row0 + kblk * DBLK, DBLK)])
        return carry
    lax.fori_loop(0, ROWS_PT // DBLK, _zy, 0)
    pltpu.sync_copy(den_v.at[pl.ds(wid * (DEN_ROWS // 16), DEN_ROWS // 16)],
                    den_sh.at[pl.ds(wid * (DEN_ROWS // 16), DEN_ROWS // 16)])
    plsc.subcore_barrier()

    def _edge(t, carry):
        off = t * 16
        src16 = src_v[pl.ds(off, 16)]
        dst16 = dst_v[pl.ds(off, 16)]
        sv = plsc.load_gather(s_v, [src16])
        dv = plsc.load_gather(d_v, [dst16])
        e = sv + dv
        e = jnp.where(e > 0.0, e, 0.2 * e)
        w16 = jnp.exp(e)
        plsc.addupdate_scatter(
            den_v,
            [lax.shift_right_logical(dst16, 7),
             lax.bitwise_and(dst16, jnp.int32(127))],
            w16)
        wbuf_v[...] = w16
        gidx = src16 + cN
        pltpu.async_copy(h2f.at[gidx], rows_v.at[0], sem_g).wait()
        for k in range(16):
            spl = lax.broadcast(wbuf_v[k], (16,))
            for j in range(8):
                rows_v[0, k, pl.ds(j * 16, 16)] = (
                    rows_v[0, k, pl.ds(j * 16, 16)] * spl)
        pltpu.async_copy(rows_v.at[0], y_sh.at[dst16], sem_s, add=True).wait()
        return carry
    lax.fori_loop(0, NBATCH, _edge, 0)
    plsc.subcore_barrier()

    pltpu.sync_copy(den_v, den_sh.at[idx80_v], add=True)
    plsc.subcore_barrier()

    pltpu.sync_copy(
        den_sh.at[pl.ds(wid * (ROWS_PT // H), ROWS_PT // H)], dch_v)

    def _div_blk(blk, carry):
        pltpu.sync_copy(y_sh.at[pl.ds(row0 + blk * DBLK, DBLK)], zbuf_v)

        def _row(r, c2):
            n = blk * DBLK + r
            wk = dch_v[lax.div(n, H), lax.rem(n, H)] + jnp.float32(1e-16)
            spl = lax.broadcast(wk, (16,))
            for j in range(8):
                zbuf_v[r, pl.ds(j * 16, 16)] = (
                    zbuf_v[r, pl.ds(j * 16, 16)] / spl)
            return c2
        lax.fori_loop(0, DBLK, _row, 0)
        pltpu.sync_copy(zbuf_v, z_hbm.at[pl.ds(cN + row0 + blk * DBLK, DBLK)])
        return carry
    lax.fori_loop(0, ROWS_PT // DBLK, _div_blk, 0)


_sc_mesh = plsc.VectorSubcoreMesh(core_axis_name="c", subcore_axis_name="s")

_gat_edges = pl.kernel(
    _gat_sc_body,
    out_type=jax.ShapeDtypeStruct((2 * NPAD, H), f32),
    mesh=_sc_mesh,
    scratch_types=[
        pltpu.VMEM((NPAD,), f32),
        pltpu.VMEM((NPAD,), f32),
        pltpu.VMEM((DEN_ROWS, H), f32),
        pltpu.VMEM((ET,), jnp.int32),
        pltpu.VMEM((ET,), jnp.int32),
        pltpu.VMEM((2, 16, H), f32),
        pltpu.VMEM((16,), f32),
        pltpu.VMEM((DEN_ROWS,), jnp.int32),
        pltpu.VMEM((ROWS_PT // H, H), f32),
        pltpu.VMEM((DBLK, H), f32),
        pltpu.VMEM_SHARED((NPAD, H), f32),
        pltpu.VMEM_SHARED((DEN_ROWS, H), f32),
        pltpu.SemaphoreType.DMA,
        pltpu.SemaphoreType.DMA,
    ],
)


def _a2p(a_src, a_dst):
    return jnp.concatenate(
        [a_src[:, None], a_dst[:, None], jnp.zeros((D, H - 2), f32)], axis=1)


def kernel(x, edge_index, W1, a_src1, a_dst1, b1,
           W2, a_src2, a_dst2, b2, W3, a_src3, a_dst3, b3):
    idt = edge_index.dtype
    loops = jnp.arange(N, dtype=idt)
    padi = jnp.full((EPAD - E2,), N, dtype=idt)
    src = jnp.concatenate([edge_index[0], loops, padi]).astype(jnp.int32)
    dst = jnp.concatenate([edge_index[1], loops, padi]).astype(jnp.int32)

    x_p = jnp.pad(x, ((0, NPAD - N), (0, 0)))

    def sc_layer(h2, sd):
        z = _gat_edges(h2.reshape(2 * NPAD, H), sd[:, 0], sd[:, 1], src, dst)
        return z.reshape(2, NPAD, H)

    h2, sd = _layer1_matmul(x_p, W1, _a2p(a_src1, a_dst1))
    z = sc_layer(h2, sd)
    h2, sd = _layer23_matmul(z, b1.reshape(2, H), W2.reshape(2, H, D),
                             _a2p(a_src2, a_dst2))
    z = sc_layer(h2, sd)
    h2, sd = _layer23_matmul(z, b2.reshape(2, H), W3.reshape(2, H, D),
                             _a2p(a_src3, a_dst3))
    z = sc_layer(h2, sd)
    out = _log_softmax(z, b3.reshape(2, H))
    return out[:N]


# speedup vs baseline: 8.2381x; 8.2381x over previous
import jax
import jax.numpy as jnp
from jax import lax
from jax.experimental import pallas as pl
from jax.experimental.pallas import tpu as pltpu
from jax.experimental.pallas import tpu_sc as plsc

N = 10000
D = 256
H = 128
NPAD = 10240
E2 = 170000
EPAD = 170240
ET = EPAD // 16
CE = 2128
ROWS_PT = NPAD // 16
DEN_ROWS = NPAD // H
NB = 512
f32 = jnp.float32


def _mm1_body(x_ref, w_ref, a2_ref, h2_ref, sd_ref):
    c = pl.program_id(1)
    h = jnp.dot(x_ref[...], w_ref[...], preferred_element_type=f32)
    h2_ref[0] = h
    part = jnp.dot(h, a2_ref[...], preferred_element_type=f32)

    @pl.when(c == 0)
    def _():
        sd_ref[...] = part

    @pl.when(c == 1)
    def _():
        sd_ref[...] = sd_ref[...] + part


def _layer1_matmul(x_p, W, A2p):
    return pl.pallas_call(
        _mm1_body,
        grid=(NPAD // NB, 2),
        in_specs=[
            pl.BlockSpec((NB, D), lambda i, c: (i, 0)),
            pl.BlockSpec((D, H), lambda i, c: (0, c)),
            pl.BlockSpec((H, H), lambda i, c: (c, 0)),
        ],
        out_specs=[
            pl.BlockSpec((1, NB, H), lambda i, c: (c, i, 0)),
            pl.BlockSpec((NB, H), lambda i, c: (i, 0)),
        ],
        out_shape=[
            jax.ShapeDtypeStruct((2, NPAD, H), f32),
            jax.ShapeDtypeStruct((NPAD, H), f32),
        ],
    )(x_p, W, A2p)


def _mm23_body(z_ref, b_ref, w_ref, a2_ref, h2_ref, sd_ref):
    c = pl.program_id(1)
    act0 = jnp.maximum(z_ref[0] + b_ref[0], 0.0)
    act1 = jnp.maximum(z_ref[1] + b_ref[1], 0.0)
    h = (jnp.dot(act0, w_ref[0], preferred_element_type=f32)
         + jnp.dot(act1, w_ref[1], preferred_element_type=f32))
    h2_ref[0] = h
    part = jnp.dot(h, a2_ref[...], preferred_element_type=f32)

    @pl.when(c == 0)
    def _():
        sd_ref[...] = part

    @pl.when(c == 1)
    def _():
        sd_ref[...] = sd_ref[...] + part


def _layer23_matmul(z, b2d, W3d, A2p):
    return pl.pallas_call(
        _mm23_body,
        grid=(NPAD // NB, 2),
        in_specs=[
            pl.BlockSpec((2, NB, H), lambda i, c: (0, i, 0)),
            pl.BlockSpec((2, H), lambda i, c: (0, 0)),
            pl.BlockSpec((2, H, H), lambda i, c: (0, 0, c)),
            pl.BlockSpec((H, H), lambda i, c: (c, 0)),
        ],
        out_specs=[
            pl.BlockSpec((1, NB, H), lambda i, c: (c, i, 0)),
            pl.BlockSpec((NB, H), lambda i, c: (i, 0)),
        ],
        out_shape=[
            jax.ShapeDtypeStruct((2, NPAD, H), f32),
            jax.ShapeDtypeStruct((NPAD, H), f32),
        ],
    )(z, b2d, W3d, A2p)


def _lsm_body(z_ref, b_ref, o_ref):
    u0 = z_ref[0] + b_ref[0]
    u1 = z_ref[1] + b_ref[1]
    u = jnp.concatenate([u0, u1], axis=1)
    m = jnp.max(u, axis=1, keepdims=True)
    lse = jnp.log(jnp.sum(jnp.exp(u - m), axis=1, keepdims=True))
    o_ref[...] = u - m - lse


def _log_softmax(z, b2d):
    return pl.pallas_call(
        _lsm_body,
        grid=(NPAD // NB,),
        in_specs=[
            pl.BlockSpec((2, NB, H), lambda i: (0, i, 0)),
            pl.BlockSpec((2, H), lambda i: (0, 0)),
        ],
        out_specs=pl.BlockSpec((NB, D), lambda i: (i, 0)),
        out_shape=jax.ShapeDtypeStruct((NPAD, D), f32),
    )(z, b2d)


def _gat_sc_body(h2f, s_hbm, d_hbm, src_hbm, dst_hbm,
                 z_hbm,
                 s_v, d_v, den_v, src_v, dst_v, rows_v, idx80_v,
                 dch_v, zbuf_v, y_sh, den_sh, sem_g, sem_s):
    cid = lax.axis_index("c")
    wid = lax.axis_index("s")
    base_e = wid * ET
    row0 = wid * ROWS_PT
    cN = cid * NPAD

    pltpu.sync_copy(s_hbm, s_v)
    pltpu.sync_copy(d_hbm, d_v)

    zeros16 = jnp.zeros((16,), f32)

    def _zden(i, carry):
        den_v[lax.div(i, 8), pl.ds(lax.rem(i, 8) * 16, 16)] = zeros16
        return carry
    lax.fori_loop(0, DEN_ROWS * 8, _zden, 0)

    def _zbuf(i, carry):
        zbuf_v[lax.div(i, 8), pl.ds(lax.rem(i, 8) * 16, 16)] = zeros16
        return carry
    lax.fori_loop(0, 16 * 8, _zbuf, 0)

    def _iot(i, carry):
        idx80_v[pl.ds(i * 16, 16)] = lax.iota(jnp.int32, 16) + i * 16
        return carry
    lax.fori_loop(0, DEN_ROWS // 16, _iot, 0)

    def _zy(kblk, carry):
        pltpu.sync_copy(zbuf_v, y_sh.at[pl.ds(row0 + kblk * 16, 16)])
        return carry
    lax.fori_loop(0, ROWS_PT // 16, _zy, 0)
    pltpu.sync_copy(den_v.at[pl.ds(wid * (DEN_ROWS // 16), DEN_ROWS // 16)],
                    den_sh.at[pl.ds(wid * (DEN_ROWS // 16), DEN_ROWS // 16)])
    plsc.subcore_barrier()

    def _chunk(ci, carry):
        pltpu.sync_copy(src_hbm.at[pl.ds(base_e + ci * CE, CE)], src_v)
        pltpu.sync_copy(dst_hbm.at[pl.ds(base_e + ci * CE, CE)], dst_v)

        def _edge(t, c2):
            off = t * 16
            src16 = src_v[pl.ds(off, 16)]
            dst16 = dst_v[pl.ds(off, 16)]
            sv = plsc.load_gather(s_v, [src16])
            dv = plsc.load_gather(d_v, [dst16])
            e = sv + dv
            e = jnp.where(e > 0.0, e, 0.2 * e)
            w16 = jnp.exp(e)
            plsc.addupdate_scatter(
                den_v,
                [lax.shift_right_logical(dst16, 7),
                 lax.bitwise_and(dst16, jnp.int32(127))],
                w16)
            gidx = src16 + cN
            pltpu.async_copy(h2f.at[gidx], rows_v.at[0], sem_g).wait()
            for k in range(16):
                spl = lax.broadcast(w16[k], (16,))
                for j in range(8):
                    rows_v[0, k, pl.ds(j * 16, 16)] = (
                        rows_v[0, k, pl.ds(j * 16, 16)] * spl)
            pltpu.async_copy(rows_v.at[0], y_sh.at[dst16], sem_s,
                             add=True).wait()
            return c2
        lax.fori_loop(0, CE // 16, _edge, 0)
        return carry
    lax.fori_loop(0, ET // CE, _chunk, 0)
    plsc.subcore_barrier()

    pltpu.sync_copy(den_v, den_sh.at[idx80_v], add=True)
    plsc.subcore_barrier()

    pltpu.sync_copy(
        den_sh.at[pl.ds(wid * (ROWS_PT // H), ROWS_PT // H)], dch_v)

    def _div_blk(g, carry):
        pltpu.sync_copy(y_sh.at[pl.ds(row0 + g * 16, 16)], zbuf_v)
        den16 = dch_v[lax.div(g, 8), pl.ds(lax.rem(g, 8) * 16, 16)]
        inv16 = jnp.float32(1.0) / (den16 + jnp.float32(1e-16))
        for r in range(16):
            spl = lax.broadcast(inv16[r], (16,))
            for j in range(8):
                zbuf_v[r, pl.ds(j * 16, 16)] = (
                    zbuf_v[r, pl.ds(j * 16, 16)] * spl)
        pltpu.sync_copy(zbuf_v, z_hbm.at[pl.ds(cN + row0 + g * 16, 16)])
        return carry
    lax.fori_loop(0, ROWS_PT // 16, _div_blk, 0)


_GAT_EDGES_CACHE = []


def _gat_edges_fn():
    if not _GAT_EDGES_CACHE:
        mesh = plsc.VectorSubcoreMesh(core_axis_name="c", subcore_axis_name="s")
        _GAT_EDGES_CACHE.append(pl.kernel(
            _gat_sc_body,
            out_type=jax.ShapeDtypeStruct((2 * NPAD, H), f32),
            mesh=mesh,
            compiler_params=pltpu.CompilerParams(needs_layout_passes=False),
            scratch_types=[
                pltpu.VMEM((NPAD,), f32),
                pltpu.VMEM((NPAD,), f32),
                pltpu.VMEM((DEN_ROWS, H), f32),
                pltpu.VMEM((CE,), jnp.int32),
                pltpu.VMEM((CE,), jnp.int32),
                pltpu.VMEM((2, 16, H), f32),
                pltpu.VMEM((DEN_ROWS,), jnp.int32),
                pltpu.VMEM((ROWS_PT // H, H), f32),
                pltpu.VMEM((16, H), f32),
                pltpu.VMEM_SHARED((NPAD, H), f32),
                pltpu.VMEM_SHARED((DEN_ROWS, H), f32),
                pltpu.SemaphoreType.DMA,
                pltpu.SemaphoreType.DMA,
            ],
        ))
    return _GAT_EDGES_CACHE[0]


def _a2p(a_src, a_dst):
    return jnp.concatenate(
        [a_src[:, None], a_dst[:, None], jnp.zeros((D, H - 2), f32)], axis=1)


def kernel(x, edge_index, W1, a_src1, a_dst1, b1,
           W2, a_src2, a_dst2, b2, W3, a_src3, a_dst3, b3):
    idt = edge_index.dtype
    loops = jnp.arange(N, dtype=idt)
    padi = jnp.full((EPAD - E2,), N, dtype=idt)
    src = jnp.concatenate([edge_index[0], loops, padi]).astype(jnp.int32)
    dst = jnp.concatenate([edge_index[1], loops, padi]).astype(jnp.int32)

    x_p = jnp.pad(x, ((0, NPAD - N), (0, 0)))

    def sc_layer(h2, sd):
        z = _gat_edges_fn()(h2.reshape(2 * NPAD, H), sd[:, 0], sd[:, 1],
                            src, dst)
        return z.reshape(2, NPAD, H)

    h2, sd = _layer1_matmul(x_p, W1, _a2p(a_src1, a_dst1))
    z = sc_layer(h2, sd)
    h2, sd = _layer23_matmul(z, b1.reshape(2, H), W2.reshape(2, H, D),
                             _a2p(a_src2, a_dst2))
    z = sc_layer(h2, sd)
    h2, sd = _layer23_matmul(z, b2.reshape(2, H), W3.reshape(2, H, D),
                             _a2p(a_src3, a_dst3))
    z = sc_layer(h2, sd)
    out = _log_softmax(z, b3.reshape(2, H))
    return out[:N]


# speedup vs baseline: 19.6311x; 2.3830x over previous
import jax
import jax.numpy as jnp
from jax import lax
from jax.experimental import pallas as pl
from jax.experimental.pallas import tpu as pltpu
from jax.experimental.pallas import tpu_sc as plsc

N = 10000
D = 256
H = 128
NPAD = 10240
E2 = 170000
EPAD = 170496
ET = EPAD // 16
CE = 1184
K = 32
NBC = CE // K
ROWS_PT = NPAD // 16
DEN_ROWS = NPAD // H
NB = 512
f32 = jnp.float32


def _mm1_body(x_ref, w_ref, a2_ref, h2_ref, sd_ref):
    c = pl.program_id(1)
    h = jnp.dot(x_ref[...], w_ref[...], preferred_element_type=f32)
    h2_ref[0] = h
    part = jnp.dot(h, a2_ref[...], preferred_element_type=f32)

    @pl.when(c == 0)
    def _():
        sd_ref[...] = part

    @pl.when(c == 1)
    def _():
        sd_ref[...] = sd_ref[...] + part


def _layer1_matmul(x_p, W, A2p):
    return pl.pallas_call(
        _mm1_body,
        grid=(NPAD // NB, 2),
        in_specs=[
            pl.BlockSpec((NB, D), lambda i, c: (i, 0)),
            pl.BlockSpec((D, H), lambda i, c: (0, c)),
            pl.BlockSpec((H, H), lambda i, c: (c, 0)),
        ],
        out_specs=[
            pl.BlockSpec((1, NB, H), lambda i, c: (c, i, 0)),
            pl.BlockSpec((NB, H), lambda i, c: (i, 0)),
        ],
        out_shape=[
            jax.ShapeDtypeStruct((2, NPAD, H), f32),
            jax.ShapeDtypeStruct((NPAD, H), f32),
        ],
    )(x_p, W, A2p)


def _mm23_body(z_ref, b_ref, w_ref, a2_ref, h2_ref, sd_ref):
    c = pl.program_id(1)
    act0 = jnp.maximum(z_ref[0] + b_ref[0], 0.0)
    act1 = jnp.maximum(z_ref[1] + b_ref[1], 0.0)
    h = (jnp.dot(act0, w_ref[0], preferred_element_type=f32)
         + jnp.dot(act1, w_ref[1], preferred_element_type=f32))
    h2_ref[0] = h
    part = jnp.dot(h, a2_ref[...], preferred_element_type=f32)

    @pl.when(c == 0)
    def _():
        sd_ref[...] = part

    @pl.when(c == 1)
    def _():
        sd_ref[...] = sd_ref[...] + part


def _layer23_matmul(z, b2d, W3d, A2p):
    return pl.pallas_call(
        _mm23_body,
        grid=(NPAD // NB, 2),
        in_specs=[
            pl.BlockSpec((2, NB, H), lambda i, c: (0, i, 0)),
            pl.BlockSpec((2, H), lambda i, c: (0, 0)),
            pl.BlockSpec((2, H, H), lambda i, c: (0, 0, c)),
            pl.BlockSpec((H, H), lambda i, c: (c, 0)),
        ],
        out_specs=[
            pl.BlockSpec((1, NB, H), lambda i, c: (c, i, 0)),
            pl.BlockSpec((NB, H), lambda i, c: (i, 0)),
        ],
        out_shape=[
            jax.ShapeDtypeStruct((2, NPAD, H), f32),
            jax.ShapeDtypeStruct((NPAD, H), f32),
        ],
    )(z, b2d, W3d, A2p)


def _lsm_body(z_ref, b_ref, o_ref):
    u0 = z_ref[0] + b_ref[0]
    u1 = z_ref[1] + b_ref[1]
    u = jnp.concatenate([u0, u1], axis=1)
    m = jnp.max(u, axis=1, keepdims=True)
    lse = jnp.log(jnp.sum(jnp.exp(u - m), axis=1, keepdims=True))
    o_ref[...] = u - m - lse


def _log_softmax(z, b2d):
    return pl.pallas_call(
        _lsm_body,
        grid=(NPAD // NB,),
        in_specs=[
            pl.BlockSpec((2, NB, H), lambda i: (0, i, 0)),
            pl.BlockSpec((2, H), lambda i: (0, 0)),
        ],
        out_specs=pl.BlockSpec((NB, D), lambda i: (i, 0)),
        out_shape=jax.ShapeDtypeStruct((NPAD, D), f32),
    )(z, b2d)


def _gat_sc_body(h2f, s_hbm, d_hbm, src_hbm, dst_hbm,
                 z_hbm,
                 s_v, d_v, den_v, src_v, dst_v, rows_v, idx80_v, dch_v,
                 y_sh, den_sh,
                 sem_g0, sem_g1, sem_g2, sem_s0, sem_s1, sem_s2):
    cid = lax.axis_index("c")
    wid = lax.axis_index("s")
    base_e = wid * ET
    row0 = wid * ROWS_PT
    cN = cid * NPAD
    sem_g = (sem_g0, sem_g1, sem_g2)
    sem_s = (sem_s0, sem_s1, sem_s2)

    pltpu.sync_copy(s_hbm, s_v)
    pltpu.sync_copy(d_hbm, d_v)

    zeros16 = jnp.zeros((16,), f32)

    def _zden(i, carry):
        den_v[lax.div(i, 8), pl.ds(lax.rem(i, 8) * 16, 16)] = zeros16
        return carry
    lax.fori_loop(0, DEN_ROWS * 8, _zden, 0)

    def _zbuf(i, carry):
        rows_v[0, lax.div(i, 8), pl.ds(lax.rem(i, 8) * 16, 16)] = zeros16
        return carry
    lax.fori_loop(0, K * 8, _zbuf, 0)

    def _iot(i, carry):
        idx80_v[pl.ds(i * 16, 16)] = lax.iota(jnp.int32, 16) + i * 16
        return carry
    lax.fori_loop(0, DEN_ROWS // 16, _iot, 0)

    def _zy(kblk, carry):
        pltpu.sync_copy(rows_v.at[0], y_sh.at[pl.ds(row0 + kblk * K, K)])
        return carry
    lax.fori_loop(0, ROWS_PT // K, _zy, 0)
    pltpu.sync_copy(den_v.at[pl.ds(wid * (DEN_ROWS // 16), DEN_ROWS // 16)],
                    den_sh.at[pl.ds(wid * (DEN_ROWS // 16), DEN_ROWS // 16)])
    plsc.subcore_barrier()

    def _fire_gather(t1, buf):
        off1 = t1 * K
        ia = src_v[pl.ds(off1, 16)] + cN
        ib = src_v[pl.ds(off1 + 16, 16)] + cN
        pltpu.async_copy(h2f.at[ia], rows_v.at[buf, pl.ds(0, 16)], sem_g[buf])
        pltpu.async_copy(h2f.at[ib], rows_v.at[buf, pl.ds(16, 16)],
                         sem_g[buf])

    def _wait(sem, buf, nrows):
        pltpu.make_async_copy(
            h2f.at[pl.ds(0, nrows)], rows_v.at[buf, pl.ds(0, nrows)],
            sem).wait()

    def _chunk(ci, carry):
        pltpu.sync_copy(src_hbm.at[pl.ds(base_e + ci * CE, CE)], src_v)
        pltpu.sync_copy(dst_hbm.at[pl.ds(base_e + ci * CE, CE)], dst_v)
        _fire_gather(0, 0)

        def _batch(t, c2):
            off = t * K
            p = lax.rem(t, 3)
            src16a = src_v[pl.ds(off, 16)]
            src16b = src_v[pl.ds(off + 16, 16)]
            dst16a = dst_v[pl.ds(off, 16)]
            dst16b = dst_v[pl.ds(off + 16, 16)]

            def _w(s16, d16):
                e = plsc.load_gather(s_v, [s16]) + plsc.load_gather(
                    d_v, [d16])
                e = jnp.where(e > 0.0, e, 0.2 * e)
                w = jnp.exp(e)
                plsc.addupdate_scatter(
                    den_v,
                    [lax.shift_right_logical(d16, 7),
                     lax.bitwise_and(d16, jnp.int32(127))],
                    w)
                return w
            w16a = _w(src16a, dst16a)
            w16b = _w(src16b, dst16b)

            for u in range(3):
                @pl.when(p == u)
                def _(u=u):
                    nxt = (u + 1) % 3

                    @pl.when((t >= 2) & (t + 1 < NBC))
                    def _():
                        _wait(sem_s[nxt], nxt, K)

                    @pl.when(t + 1 < NBC)
                    def _():
                        _fire_gather(t + 1, nxt)
                    _wait(sem_g[u], u, K)

            for k in range(16):
                spla = lax.broadcast(w16a[k], (16,))
                splb = lax.broadcast(w16b[k], (16,))
                for j in range(8):
                    rows_v[p, k, pl.ds(j * 16, 16)] = (
                        rows_v[p, k, pl.ds(j * 16, 16)] * spla)
                    rows_v[p, k + 16, pl.ds(j * 16, 16)] = (
                        rows_v[p, k + 16, pl.ds(j * 16, 16)] * splb)

            for u in range(3):
                @pl.when(p == u)
                def _(u=u):
                    pltpu.async_copy(rows_v.at[u, pl.ds(0, 16)],
                                     y_sh.at[dst16a], sem_s[u], add=True)
                    pltpu.async_copy(rows_v.at[u, pl.ds(16, 16)],
                                     y_sh.at[dst16b], sem_s[u], add=True)
            return c2
        lax.fori_loop(0, NBC, _batch, 0)

        for u in range(3):
            _wait(sem_s[u], u, K)
        return carry
    lax.fori_loop(0, ET // CE, _chunk, 0)
    plsc.subcore_barrier()

    pltpu.sync_copy(den_v, den_sh.at[idx80_v], add=True)
    plsc.subcore_barrier()

    pltpu.sync_copy(
        den_sh.at[pl.ds(wid * (ROWS_PT // H), ROWS_PT // H)], dch_v)

    def _div_blk(g, carry):
        pltpu.sync_copy(y_sh.at[pl.ds(row0 + g * K, K)], rows_v.at[0])
        dr = lax.div(g, 4)
        do = lax.rem(g, 4) * K
        inva = jnp.float32(1.0) / (dch_v[dr, pl.ds(do, 16)]
                                   + jnp.float32(1e-16))
        invb = jnp.float32(1.0) / (dch_v[dr, pl.ds(do + 16, 16)]
                                   + jnp.float32(1e-16))
        for r in range(16):
            spla = lax.broadcast(inva[r], (16,))
            splb = lax.broadcast(invb[r], (16,))
            for j in range(8):
                rows_v[0, r, pl.ds(j * 16, 16)] = (
                    rows_v[0, r, pl.ds(j * 16, 16)] * spla)
                rows_v[0, r + 16, pl.ds(j * 16, 16)] = (
                    rows_v[0, r + 16, pl.ds(j * 16, 16)] * splb)
        pltpu.sync_copy(rows_v.at[0], z_hbm.at[pl.ds(cN + row0 + g * K, K)])
        return carry
    lax.fori_loop(0, ROWS_PT // K, _div_blk, 0)


_GAT_EDGES_CACHE = []


def _gat_edges_fn():
    if not _GAT_EDGES_CACHE:
        mesh = plsc.VectorSubcoreMesh(core_axis_name="c", subcore_axis_name="s")
        _GAT_EDGES_CACHE.append(pl.kernel(
            _gat_sc_body,
            out_type=jax.ShapeDtypeStruct((2 * NPAD, H), f32),
            mesh=mesh,
            compiler_params=pltpu.CompilerParams(needs_layout_passes=False),
            scratch_types=[
                pltpu.VMEM((NPAD,), f32),
                pltpu.VMEM((NPAD,), f32),
                pltpu.VMEM((DEN_ROWS, H), f32),
                pltpu.VMEM((CE,), jnp.int32),
                pltpu.VMEM((CE,), jnp.int32),
                pltpu.VMEM((3, K, H), f32),
                pltpu.VMEM((DEN_ROWS,), jnp.int32),
                pltpu.VMEM((ROWS_PT // H, H), f32),
                pltpu.VMEM_SHARED((NPAD, H), f32),
                pltpu.VMEM_SHARED((DEN_ROWS, H), f32),
                pltpu.SemaphoreType.DMA,
                pltpu.SemaphoreType.DMA,
                pltpu.SemaphoreType.DMA,
                pltpu.SemaphoreType.DMA,
                pltpu.SemaphoreType.DMA,
                pltpu.SemaphoreType.DMA,
            ],
        ))
    return _GAT_EDGES_CACHE[0]


def _a2p(a_src, a_dst):
    return jnp.concatenate(
        [a_src[:, None], a_dst[:, None], jnp.zeros((D, H - 2), f32)], axis=1)


def kernel(x, edge_index, W1, a_src1, a_dst1, b1,
           W2, a_src2, a_dst2, b2, W3, a_src3, a_dst3, b3):
    idt = edge_index.dtype
    loops = jnp.arange(N, dtype=idt)
    padi = jnp.full((EPAD - E2,), N, dtype=idt)
    src = jnp.concatenate([edge_index[0], loops, padi]).astype(jnp.int32)
    dst = jnp.concatenate([edge_index[1], loops, padi]).astype(jnp.int32)

    x_p = jnp.pad(x, ((0, NPAD - N), (0, 0)))

    def sc_layer(h2, sd):
        z = _gat_edges_fn()(h2.reshape(2 * NPAD, H), sd[:, 0], sd[:, 1],
                            src, dst)
        return z.reshape(2, NPAD, H)

    h2, sd = _layer1_matmul(x_p, W1, _a2p(a_src1, a_dst1))
    z = sc_layer(h2, sd)
    h2, sd = _layer23_matmul(z, b1.reshape(2, H), W2.reshape(2, H, D),
                             _a2p(a_src2, a_dst2))
    z = sc_layer(h2, sd)
    h2, sd = _layer23_matmul(z, b2.reshape(2, H), W3.reshape(2, H, D),
                             _a2p(a_src3, a_dst3))
    z = sc_layer(h2, sd)
    out = _log_softmax(z, b3.reshape(2, H))
    return out[:N]
